# baseline (speedup 1.0000x reference)
import functools

import jax
import jax.numpy as jnp
from jax import lax
from jax.experimental import pallas as pl
from jax.experimental.pallas import tpu as pltpu
from jax.experimental.pallas import tpu_sc as plsc

_NB = 4
_NP = 2
_R = 16
_NW = 32
_SEQ_BLK = 2048


def _add_body(x_ref, pe_ref, o_ref):
    o_ref[...] = x_ref[...] + pe_ref[...]


def _tc_add(x, pe_weight, nbatch):
    batch, seq, d = x.shape
    grid = (seq // _SEQ_BLK, nbatch)
    return pl.pallas_call(
        _add_body,
        grid=grid,
        in_specs=[
            pl.BlockSpec((1, _SEQ_BLK, d), lambda s, b: (b, s, 0)),
            pl.BlockSpec((_SEQ_BLK, d), lambda s, b: (s, 0)),
        ],
        out_specs=pl.BlockSpec((1, _SEQ_BLK, d), lambda s, b: (b, s, 0)),
        out_shape=jax.ShapeDtypeStruct((nbatch, seq, d), x.dtype),
        compiler_params=pltpu.CompilerParams(
            dimension_semantics=("arbitrary", "arbitrary"),
        ),
    )(x, pe_weight)


def _sc_body(seq, d, chunk, row_lo, x_hbm, pe_hbm, out_hbm, xbufs, pbufs, ibufs, sx, sg, so):
    nt = chunk // _R
    c = lax.axis_index("c")
    s = lax.axis_index("s")
    wid = s * 2 + c
    obase = wid * chunk
    base = row_lo + obase
    pe_base = lax.rem(base, seq)

    def start_load(t, b):
        pltpu.async_copy(x_hbm.at[pl.ds(base + t * _R, _R)], xbufs[b], sx[b])

    def wait_load(b):
        pltpu.make_async_copy(x_hbm.at[pl.ds(0, _R)], xbufs[b], sx[b]).wait()

    def start_gather(t, p):
        ibufs[p][...] = lax.iota(jnp.int32, _R) + (pe_base + t * _R)
        pltpu.async_copy(pe_hbm.at[ibufs[p]], pbufs[p], sg[p])

    def wait_gather(p):
        pltpu.make_async_copy(pe_hbm.at[ibufs[p]], pbufs[p], sg[p]).wait()

    def add_tile(b, p):
        xb, pb = xbufs[b], pbufs[p]

        def row(r, carry):
            for c2 in range(d // 16):
                sl = (r, pl.ds(c2 * 16, 16))
                xb[sl] = xb[sl] + pb[sl]
            return carry

        lax.fori_loop(0, _R, row, None)

    def start_store(t, b):
        pltpu.async_copy(xbufs[b], out_hbm.at[pl.ds(obase + t * _R, _R)], so[b])

    def wait_store(b):
        pltpu.make_async_copy(xbufs[b], out_hbm.at[pl.ds(0, _R)], so[b]).wait()

    start_load(0, 0)
    start_load(1, 1)
    wait_load(0)
    start_gather(0, 0)
    start_load(2, 2)
    wait_load(1)
    start_gather(1, 1)
    wait_gather(0)
    add_tile(0, 0)
    start_store(0, 0)
    start_load(3, 3)
    wait_load(2)
    start_gather(2, 0)
    wait_gather(1)
    add_tile(1, 1)
    start_store(1, 1)

    def outer(j, carry):
        t0 = 4 + j * _NB
        for i in range(_NB):
            t = t0 + i
            wait_store(i)
            start_load(t, i)
            wait_load((i - 1) % _NB)
            start_gather(t - 1, (i - 1) % _NP)
            wait_gather(i % _NP)
            add_tile((i - 2) % _NB, i % _NP)
            start_store(t - 2, (i - 2) % _NB)
        return carry

    lax.fori_loop(0, (nt - 4) // _NB, outer, None)

    wait_load((nt - 1) % _NB)
    start_gather(nt - 1, (nt - 1) % _NP)
    wait_gather((nt - 2) % _NP)
    add_tile((nt - 2) % _NB, (nt - 2) % _NP)
    start_store(nt - 2, (nt - 2) % _NB)
    wait_gather((nt - 1) % _NP)
    add_tile((nt - 1) % _NB, (nt - 1) % _NP)
    start_store(nt - 1, (nt - 1) % _NB)
    for b in range(_NB):
        wait_store(b)


def _sc_add(x2d, pe_weight, row_lo):
    rows, d = x2d.shape
    seq = pe_weight.shape[0]
    nrows = rows - row_lo
    chunk = nrows // _NW
    body = functools.partial(_sc_body, seq, d, chunk, row_lo)
    fn = pl.kernel(
        body,
        out_type=jax.ShapeDtypeStruct((nrows, d), jnp.float32),
        mesh=plsc.VectorSubcoreMesh(core_axis_name="c", subcore_axis_name="s"),
        scratch_types=[
            [pltpu.VMEM((_R, d), jnp.float32) for _ in range(_NB)],
            [pltpu.VMEM((_R, d), jnp.float32) for _ in range(_NP)],
            [pltpu.VMEM((_R,), jnp.int32) for _ in range(_NP)],
            [pltpu.SemaphoreType.DMA for _ in range(_NB)],
            [pltpu.SemaphoreType.DMA for _ in range(_NP)],
            [pltpu.SemaphoreType.DMA for _ in range(_NB)],
        ],
    )
    return fn(x2d, pe_weight)


def kernel(x, pe_weight):
    batch, seq, d = x.shape
    tc_out = _tc_add(x, pe_weight, batch - 1)
    sc_out = _sc_add(x.reshape(batch * seq, d), pe_weight, (batch - 1) * seq)
    return jnp.concatenate([tc_out, sc_out.reshape(1, seq, d)], axis=0)


# speedup vs baseline: 2.2225x; 2.2225x over previous
import jax
import jax.numpy as jnp
from jax.experimental import pallas as pl
from jax.experimental.pallas import tpu as pltpu

_SEQ_BLK = 2048


def _add_body(x_ref, pe_ref, o_ref):
    o_ref[...] = x_ref[...] + pe_ref[...]


def kernel(x, pe_weight):
    batch, seq, d = x.shape
    grid = (seq // _SEQ_BLK, batch)
    return pl.pallas_call(
        _add_body,
        grid=grid,
        in_specs=[
            pl.BlockSpec((1, _SEQ_BLK, d), lambda s, b: (b, s, 0)),
            pl.BlockSpec((_SEQ_BLK, d), lambda s, b: (s, 0)),
        ],
        out_specs=pl.BlockSpec((1, _SEQ_BLK, d), lambda s, b: (b, s, 0)),
        out_shape=jax.ShapeDtypeStruct((batch, seq, d), x.dtype),
        compiler_params=pltpu.CompilerParams(
            dimension_semantics=("arbitrary", "arbitrary"),
        ),
    )(x, pe_weight)
